# Initial kernel scaffold; baseline (speedup 1.0000x reference)
import jax
import jax.numpy as jnp
from jax.experimental import pallas as pl


def kernel(descriptors0, descriptors1):
    raise NotImplementedError("write your pallas kernel here")


# speedup vs baseline: 86.1696x; 86.1696x over previous
import functools

import jax
import jax.numpy as jnp
from jax import lax
from jax.experimental import pallas as pl
from jax.experimental.pallas import tpu as pltpu


def _nn_body(d0_ref, d1_ref, m_ref, s_ref, m1s_ref, *, n, m, t, ratio2):
    d0 = d0_ref[0]
    nt = m // t
    lane_n = lax.broadcasted_iota(jnp.int32, (1, n), 1)
    neg_inf = jnp.float32(-jnp.inf)

    def loop1(mt, carry):
        v1, i1, v2 = carry
        a = d1_ref[0, :, pl.ds(mt * t, t)]
        st = lax.dot_general(a, d0, (((0,), (0,)), ((), ())),
                             preferred_element_type=jnp.float32)
        lane = lax.broadcasted_iota(jnp.int32, (t, n), 1)
        sub = lax.broadcasted_iota(jnp.int32, (t, n), 0)
        u1 = jnp.max(st, axis=1, keepdims=True)
        jrel = jnp.min(jnp.where(st == u1, lane, n), axis=1, keepdims=True)
        u2 = jnp.max(jnp.where(lane == jrel, neg_inf, st), axis=1, keepdims=True)
        okk = (2.0 * (1.0 - u1)) <= ratio2 * (2.0 * (1.0 - u2))
        m1s_ref[pl.ds(mt * t, t), :] = jnp.where(okk, jrel, -1).astype(jnp.int32)
        t1 = jnp.max(st, axis=0, keepdims=True)
        trel = jnp.min(jnp.where(st == t1, sub, t), axis=0, keepdims=True)
        t2 = jnp.max(jnp.where(sub == trel, neg_inf, st), axis=0, keepdims=True)
        tg = trel + mt * t
        i1n = jnp.where(t1 > v1, tg, i1)
        v2n = jnp.maximum(jnp.minimum(v1, t1), jnp.maximum(v2, t2))
        v1n = jnp.maximum(v1, t1)
        return v1n, i1n, v2n

    init = (jnp.full((1, n), neg_inf), jnp.zeros((1, n), jnp.int32),
            jnp.full((1, n), neg_inf))
    v1, i1, v2 = lax.fori_loop(0, nt, loop1, init, unroll=True)

    maskq = (2.0 * (1.0 - v1)) <= ratio2 * (2.0 * (1.0 - v2))
    m0 = jnp.where(maskq, i1, -1).astype(jnp.int32)
    scores = jnp.where(maskq, (v1 + 1.0) / 2.0, 0.0)

    def loop2(mt, ok):
        m1t = m1s_ref[pl.ds(mt * t, t), :]
        jg = lax.broadcasted_iota(jnp.int32, (t, 1), 0) + mt * t
        hit = jnp.any((m0 == jg) & (m1t == lane_n), axis=0, keepdims=True)
        return ok | hit

    ok = lax.fori_loop(0, nt, loop2, jnp.zeros((1, n), jnp.bool_), unroll=True)
    m_ref[0] = jnp.where(ok, m0, -1)
    s_ref[0] = scores.astype(jnp.float32)


def kernel(descriptors0, descriptors1):
    b, d, n = descriptors0.shape
    m = descriptors1.shape[2]
    t = 256
    body = functools.partial(_nn_body, n=n, m=m, t=t,
                             ratio2=0.8 ** 2)
    matches3, scores3 = pl.pallas_call(
        body,
        grid=(b,),
        in_specs=[pl.BlockSpec((1, d, n), lambda i: (i, 0, 0)),
                  pl.BlockSpec((1, d, m), lambda i: (i, 0, 0))],
        out_specs=[pl.BlockSpec((1, 1, n), lambda i: (i, 0, 0)),
                   pl.BlockSpec((1, 1, n), lambda i: (i, 0, 0))],
        out_shape=[jax.ShapeDtypeStruct((b, 1, n), jnp.int32),
                   jax.ShapeDtypeStruct((b, 1, n), jnp.float32)],
        scratch_shapes=[pltpu.VMEM((m, 1), jnp.int32)],
        compiler_params=pltpu.CompilerParams(
            dimension_semantics=("parallel",)),
    )(descriptors0, descriptors1)
    return matches3.reshape(b, n), scores3.reshape(b, n)


# speedup vs baseline: 97.2354x; 1.1284x over previous
import functools

import jax
import jax.numpy as jnp
from jax import lax
from jax.experimental import pallas as pl
from jax.experimental.pallas import tpu as pltpu


def _nn_body(d0_ref, d1_ref, m_ref, s_ref, m1s_ref, *, n, m, t, ratio2):
    d0 = d0_ref[0]
    nt = m // t
    lane_n = lax.broadcasted_iota(jnp.int32, (1, n), 1)
    neg_inf = jnp.float32(-jnp.inf)

    def loop1(mt, carry):
        v1, i1, v2 = carry
        a = d1_ref[0, :, pl.ds(mt * t, t)]
        st = lax.dot_general(a, d0, (((0,), (0,)), ((), ())),
                             preferred_element_type=jnp.float32)
        lane = lax.broadcasted_iota(jnp.int32, (t, n), 1)
        sub = lax.broadcasted_iota(jnp.int32, (t, n), 0)
        u1 = jnp.max(st, axis=1, keepdims=True)
        jrel = jnp.min(jnp.where(st == u1, lane, n), axis=1, keepdims=True)
        u2 = jnp.max(jnp.where(lane == jrel, neg_inf, st), axis=1, keepdims=True)
        okk = (2.0 * (1.0 - u1)) <= ratio2 * (2.0 * (1.0 - u2))
        m1s_ref[pl.ds(mt * t, t), :] = jnp.where(okk, jrel, -1).astype(jnp.int32)
        t1 = jnp.max(st, axis=0, keepdims=True)
        trel = jnp.min(jnp.where(st == t1, sub, t), axis=0, keepdims=True)
        t2 = jnp.max(jnp.where(sub == trel, neg_inf, st), axis=0, keepdims=True)
        tg = trel + mt * t
        i1n = jnp.where(t1 > v1, tg, i1)
        v2n = jnp.maximum(jnp.minimum(v1, t1), jnp.maximum(v2, t2))
        v1n = jnp.maximum(v1, t1)
        return v1n, i1n, v2n

    init = (jnp.full((1, n), neg_inf), jnp.zeros((1, n), jnp.int32),
            jnp.full((1, n), neg_inf))
    v1, i1, v2 = lax.fori_loop(0, nt, loop1, init, unroll=True)

    maskq = (2.0 * (1.0 - v1)) <= ratio2 * (2.0 * (1.0 - v2))
    m0 = jnp.where(maskq, i1, -1).astype(jnp.int32)
    scores = jnp.where(maskq, (v1 + 1.0) / 2.0, 0.0)

    target = lane_n * m + m0

    def loop2(mt, ok):
        m1t = m1s_ref[pl.ds(mt * t, t), :]
        jg = lax.broadcasted_iota(jnp.int32, (t, 1), 0) + mt * t
        ckey = m1t * m + jg
        hit = jnp.any(ckey == target, axis=0, keepdims=True)
        return ok | hit

    ok = lax.fori_loop(0, nt, loop2, jnp.zeros((1, n), jnp.bool_), unroll=True)
    m_ref[0] = jnp.where(ok, m0, -1)
    s_ref[0] = scores.astype(jnp.float32)


def kernel(descriptors0, descriptors1):
    b, d, n = descriptors0.shape
    m = descriptors1.shape[2]
    t = 256
    body = functools.partial(_nn_body, n=n, m=m, t=t,
                             ratio2=0.8 ** 2)
    matches3, scores3 = pl.pallas_call(
        body,
        grid=(b,),
        in_specs=[pl.BlockSpec((1, d, n), lambda i: (i, 0, 0)),
                  pl.BlockSpec((1, d, m), lambda i: (i, 0, 0))],
        out_specs=[pl.BlockSpec((1, 1, n), lambda i: (i, 0, 0)),
                   pl.BlockSpec((1, 1, n), lambda i: (i, 0, 0))],
        out_shape=[jax.ShapeDtypeStruct((b, 1, n), jnp.int32),
                   jax.ShapeDtypeStruct((b, 1, n), jnp.float32)],
        scratch_shapes=[pltpu.VMEM((m, 1), jnp.int32)],
        compiler_params=pltpu.CompilerParams(
            dimension_semantics=("parallel",)),
    )(descriptors0, descriptors1)
    return matches3.reshape(b, n), scores3.reshape(b, n)


# speedup vs baseline: 120.2778x; 1.2370x over previous
import functools

import jax
import jax.numpy as jnp
from jax import lax
from jax.experimental import pallas as pl
from jax.experimental.pallas import tpu as pltpu


def _nn_body(d0_ref, d1_ref, m_ref, s_ref, m1s_ref, *, n, m, t, ratio2):
    d0 = d0_ref[0]
    nt = m // t
    lane_n = lax.broadcasted_iota(jnp.int32, (1, n), 1)
    neg_inf = jnp.float32(-jnp.inf)

    def loop1(mt, carry):
        v1, i1, v2 = carry
        a = d1_ref[0, :, pl.ds(mt * t, t)]
        st = lax.dot_general(a, d0, (((0,), (0,)), ((), ())),
                             preferred_element_type=jnp.float32)
        lane = lax.broadcasted_iota(jnp.int32, (t, n), 1)
        u1 = jnp.max(st, axis=1, keepdims=True)
        jrel = jnp.min(jnp.where(st == u1, lane, n), axis=1, keepdims=True)
        u2 = jnp.max(jnp.where(lane == jrel, neg_inf, st), axis=1, keepdims=True)
        okk = (2.0 * (1.0 - u1)) <= ratio2 * (2.0 * (1.0 - u2))
        m1s_ref[pl.ds(mt * t, t), :] = jnp.where(okk, jrel, -1).astype(jnp.int32)
        cv1 = jnp.full((8, n), neg_inf)
        ci1 = jnp.zeros((8, n), jnp.int32)
        cv2 = jnp.full((8, n), neg_inf)
        for c in range(t // 8):
            s = st[c * 8:(c + 1) * 8, :]
            gt = s > cv1
            cv2 = jnp.maximum(cv2, jnp.minimum(cv1, s))
            cv1 = jnp.maximum(cv1, s)
            ci1 = jnp.where(gt, c, ci1)
        sub8 = lax.broadcasted_iota(jnp.int32, (8, n), 0)
        g1 = ci1 * 8 + sub8
        t1 = jnp.max(cv1, axis=0, keepdims=True)
        trel = jnp.min(jnp.where(cv1 == t1, g1, t), axis=0, keepdims=True)
        rstar = jnp.bitwise_and(trel, 7)
        v2c = jnp.max(jnp.where(sub8 == rstar, neg_inf, cv1), axis=0,
                      keepdims=True)
        t2 = jnp.maximum(v2c, jnp.max(cv2, axis=0, keepdims=True))
        tg = trel + mt * t
        i1n = jnp.where(t1 > v1, tg, i1)
        v2n = jnp.maximum(jnp.minimum(v1, t1), jnp.maximum(v2, t2))
        v1n = jnp.maximum(v1, t1)
        return v1n, i1n, v2n

    init = (jnp.full((1, n), neg_inf), jnp.zeros((1, n), jnp.int32),
            jnp.full((1, n), neg_inf))
    v1, i1, v2 = lax.fori_loop(0, nt, loop1, init, unroll=True)

    maskq = (2.0 * (1.0 - v1)) <= ratio2 * (2.0 * (1.0 - v2))
    m0 = jnp.where(maskq, i1, -1).astype(jnp.int32)
    scores = jnp.where(maskq, (v1 + 1.0) / 2.0, 0.0)

    target = lane_n * m + m0

    def loop2(mt, ok):
        m1t = m1s_ref[pl.ds(mt * t, t), :]
        jg = lax.broadcasted_iota(jnp.int32, (t, 1), 0) + mt * t
        ckey = m1t * m + jg
        hit = jnp.any(ckey == target, axis=0, keepdims=True)
        return ok | hit

    ok = lax.fori_loop(0, nt, loop2, jnp.zeros((1, n), jnp.bool_), unroll=True)
    m_ref[0] = jnp.where(ok, m0, -1)
    s_ref[0] = scores.astype(jnp.float32)


def kernel(descriptors0, descriptors1):
    b, d, n = descriptors0.shape
    m = descriptors1.shape[2]
    t = 256
    body = functools.partial(_nn_body, n=n, m=m, t=t,
                             ratio2=0.8 ** 2)
    matches3, scores3 = pl.pallas_call(
        body,
        grid=(b,),
        in_specs=[pl.BlockSpec((1, d, n), lambda i: (i, 0, 0)),
                  pl.BlockSpec((1, d, m), lambda i: (i, 0, 0))],
        out_specs=[pl.BlockSpec((1, 1, n), lambda i: (i, 0, 0)),
                   pl.BlockSpec((1, 1, n), lambda i: (i, 0, 0))],
        out_shape=[jax.ShapeDtypeStruct((b, 1, n), jnp.int32),
                   jax.ShapeDtypeStruct((b, 1, n), jnp.float32)],
        scratch_shapes=[pltpu.VMEM((m, 1), jnp.int32)],
        compiler_params=pltpu.CompilerParams(
            dimension_semantics=("parallel",)),
    )(descriptors0, descriptors1)
    return matches3.reshape(b, n), scores3.reshape(b, n)


# speedup vs baseline: 129.8989x; 1.0800x over previous
import functools

import jax
import jax.numpy as jnp
from jax import lax
from jax.experimental import pallas as pl
from jax.experimental.pallas import tpu as pltpu


def _tile_top2(st, t, base, neg_inf):
    w = st.shape[1]
    cv1 = jnp.full((8, w), neg_inf)
    ci1 = jnp.zeros((8, w), jnp.int32)
    cv2 = jnp.full((8, w), neg_inf)
    for c in range(t // 8):
        s = st[c * 8:(c + 1) * 8, :]
        gt = s > cv1
        cv2 = jnp.maximum(cv2, jnp.minimum(cv1, s))
        cv1 = jnp.maximum(cv1, s)
        ci1 = jnp.where(gt, c, ci1)
    sub8 = lax.broadcasted_iota(jnp.int32, (8, w), 0)
    g1 = ci1 * 8 + sub8
    t1 = jnp.max(cv1, axis=0, keepdims=True)
    trel = jnp.min(jnp.where(cv1 == t1, g1, t), axis=0, keepdims=True)
    rstar = jnp.bitwise_and(trel, 7)
    v2c = jnp.max(jnp.where(sub8 == rstar, neg_inf, cv1), axis=0,
                  keepdims=True)
    t2 = jnp.maximum(v2c, jnp.max(cv2, axis=0, keepdims=True))
    return t1, trel + base, t2


def _merge_top2(run, tile):
    v1, i1, v2 = run
    t1, tg, t2 = tile
    i1n = jnp.where(t1 > v1, tg, i1)
    v2n = jnp.maximum(jnp.minimum(v1, t1), jnp.maximum(v2, t2))
    v1n = jnp.maximum(v1, t1)
    return v1n, i1n, v2n


def _nn_body(d0_ref, d1_ref, m_ref, s_ref, *, n, m, t, ratio2):
    d0 = d0_ref[0]
    d1 = d1_ref[0]
    nt = m // t
    neg_inf = jnp.float32(-jnp.inf)
    lane_n = lax.broadcasted_iota(jnp.int32, (1, n), 1)
    lane_m = lax.broadcasted_iota(jnp.int32, (1, m), 1)

    qrun = (jnp.full((1, n), neg_inf), jnp.zeros((1, n), jnp.int32),
            jnp.full((1, n), neg_inf))
    krun = (jnp.full((1, m), neg_inf), jnp.zeros((1, m), jnp.int32),
            jnp.full((1, m), neg_inf))
    for mt in range(nt):
        a = d1[:, mt * t:(mt + 1) * t]
        st1 = lax.dot_general(a, d0, (((0,), (0,)), ((), ())),
                              preferred_element_type=jnp.float32)
        qrun = _merge_top2(qrun, _tile_top2(st1, t, mt * t, neg_inf))
        b = d0[:, mt * t:(mt + 1) * t]
        st2 = lax.dot_general(b, d1, (((0,), (0,)), ((), ())),
                              preferred_element_type=jnp.float32)
        krun = _merge_top2(krun, _tile_top2(st2, t, mt * t, neg_inf))

    v1, i1, v2 = qrun
    maskq = (2.0 * (1.0 - v1)) <= ratio2 * (2.0 * (1.0 - v2))
    m0 = jnp.where(maskq, i1, -1).astype(jnp.int32)
    scores = jnp.where(maskq, (v1 + 1.0) / 2.0, 0.0)

    w1, j1, w2 = krun
    maskk = (2.0 * (1.0 - w1)) <= ratio2 * (2.0 * (1.0 - w2))
    m1 = jnp.where(maskk, j1, -1).astype(jnp.int32)

    ckey = (m1 * m + lane_m).reshape(m, 1)
    target = lane_n * m + m0
    ok = jnp.zeros((1, n), jnp.bool_)
    for mt in range(nt):
        ck = ckey[mt * t:(mt + 1) * t, :]
        ok = ok | jnp.any(ck == target, axis=0, keepdims=True)

    m_ref[0] = jnp.where(ok, m0, -1)
    s_ref[0] = scores.astype(jnp.float32)


def kernel(descriptors0, descriptors1):
    b, d, n = descriptors0.shape
    m = descriptors1.shape[2]
    t = 256
    body = functools.partial(_nn_body, n=n, m=m, t=t,
                             ratio2=0.8 ** 2)
    matches3, scores3 = pl.pallas_call(
        body,
        grid=(b,),
        in_specs=[pl.BlockSpec((1, d, n), lambda i: (i, 0, 0)),
                  pl.BlockSpec((1, d, m), lambda i: (i, 0, 0))],
        out_specs=[pl.BlockSpec((1, 1, n), lambda i: (i, 0, 0)),
                   pl.BlockSpec((1, 1, n), lambda i: (i, 0, 0))],
        out_shape=[jax.ShapeDtypeStruct((b, 1, n), jnp.int32),
                   jax.ShapeDtypeStruct((b, 1, n), jnp.float32)],
        compiler_params=pltpu.CompilerParams(
            dimension_semantics=("parallel",)),
    )(descriptors0, descriptors1)
    return matches3.reshape(b, n), scores3.reshape(b, n)
